# Initial kernel scaffold; baseline (speedup 1.0000x reference)
import jax
import jax.numpy as jnp
from jax.experimental import pallas as pl


def kernel(x, edge_index, edge_attr, W1, b1, W2, b2):
    raise NotImplementedError("write your pallas kernel here")


# speedup vs baseline: 6.3750x; 6.3750x over previous
import functools

import jax
import jax.numpy as jnp
from jax import lax
from jax.experimental import pallas as pl
from jax.experimental.pallas import tpu as pltpu
from jax.experimental.pallas import tpu_sc as plsc

N, E, FX, FE, H, FO = 10000, 320000, 128, 16, 128, 128
NC, NS = 2, 16
NW = NC * NS
EPT = E // NW
CHUNK = 2000
NCHUNK = EPT // CHUNK
ROWS = 624
ZROWS = ROWS


def _sc_entry(dst_hbm, attr_hbm, ones_hbm, zeros_hbm, sums_out, cnts_out,
              acc_s, acc_c, attr_v, ones_v, *idx_vs):
    if True:
        c = lax.axis_index("c")
        s = lax.axis_index("s")
        wid = c * NS + s

        pltpu.sync_copy(ones_hbm, ones_v)
        ebase = wid * EPT
        for j in range(NCHUNK):
            pltpu.sync_copy(dst_hbm.at[pl.ds(ebase + j * CHUNK, CHUNK)],
                            idx_vs[j])

        row0 = s * ROWS
        pltpu.sync_copy(zeros_hbm, acc_s.at[pl.ds(row0, ROWS)])
        pltpu.sync_copy(zeros_hbm, acc_c.at[pl.ds(row0, ROWS)])

        tail0, tail = NS * ROWS, N - NS * ROWS

        @pl.when(s == NS - 1)
        def _():
            pltpu.sync_copy(zeros_hbm.at[pl.ds(0, tail)],
                            acc_s.at[pl.ds(tail0, tail)])
            pltpu.sync_copy(zeros_hbm.at[pl.ds(0, tail)],
                            acc_c.at[pl.ds(tail0, tail)])

        plsc.subcore_barrier()

        for j in range(NCHUNK):
            pltpu.sync_copy(attr_hbm.at[pl.ds(ebase + j * CHUNK, CHUNK)],
                            attr_v)
            pltpu.sync_copy(attr_v, acc_s.at[idx_vs[j]], add=True)
            pltpu.sync_copy(ones_v, acc_c.at[idx_vs[j]], add=True)

        plsc.subcore_barrier()

        pltpu.sync_copy(acc_s.at[pl.ds(row0, ROWS)],
                        sums_out.at[c, pl.ds(row0, ROWS)])
        pltpu.sync_copy(acc_c.at[pl.ds(row0, ROWS)],
                        cnts_out.at[c, pl.ds(row0, ROWS)])

        @pl.when(s == NS - 1)
        def _():
            pltpu.sync_copy(acc_s.at[pl.ds(tail0, tail)],
                            sums_out.at[c, pl.ds(tail0, tail)])
            pltpu.sync_copy(acc_c.at[pl.ds(tail0, tail)],
                            cnts_out.at[c, pl.ds(tail0, tail)])


@jax.jit
def _sc_scatter(dst, attr, ones, zeros):
    mesh = plsc.VectorSubcoreMesh(core_axis_name="c", subcore_axis_name="s")
    f = pl.kernel(
        _sc_entry,
        out_type=[jax.ShapeDtypeStruct((NC, N, FE), jnp.float32),
                  jax.ShapeDtypeStruct((NC, N, FE), jnp.float32)],
        mesh=mesh,
        scratch_types=(
            [pltpu.VMEM_SHARED((N, FE), jnp.float32),
             pltpu.VMEM_SHARED((N, FE), jnp.float32),
             pltpu.VMEM((CHUNK, FE), jnp.float32),
             pltpu.VMEM((CHUNK, FE), jnp.float32)]
            + [pltpu.VMEM((CHUNK,), jnp.int32) for _ in range(NCHUNK)]
        ),
        compiler_params=pltpu.CompilerParams(use_tc_tiling_on_sc=False),
        name="sc_scatter_mean",
    )
    return f(dst, attr, ones, zeros)


def _tc_body(x_ref, s_ref, c_ref, w1a_ref, w1b_ref, b1_ref, w2_ref, b2_ref,
             o_ref):
    ssum = s_ref[0] + s_ref[1]
    cnt = jnp.maximum(c_ref[0] + c_ref[1], 1.0)
    agg = ssum / cnt
    h = jnp.dot(x_ref[...], w1a_ref[...], preferred_element_type=jnp.float32)
    h += jnp.dot(agg, w1b_ref[...], preferred_element_type=jnp.float32)
    h = jnp.maximum(h + b1_ref[...], 0.0)
    o_ref[...] = (jnp.dot(h, w2_ref[...], preferred_element_type=jnp.float32)
                  + b2_ref[...])


@jax.jit
def _tc_mlp(x, sums, cnts, W1a, W1b, b1, W2, b2):
    return pl.pallas_call(
        _tc_body,
        out_shape=jax.ShapeDtypeStruct((N, FO), jnp.float32),
        name="tc_node_mlp",
    )(x, sums, cnts, W1a, W1b, b1, W2, b2)


def kernel(x, edge_index, edge_attr, W1, b1, W2, b2):
    ones = jnp.ones((CHUNK, FE), jnp.float32)
    zeros = jnp.zeros((ZROWS, FE), jnp.float32)
    sums, cnts = _sc_scatter(edge_index[1], edge_attr, ones, zeros)
    return _tc_mlp(x, sums, cnts, W1[:FX], W1[FX:], b1.reshape(1, H),
                   W2, b2.reshape(1, FO))
